# baseline (speedup 1.0000x reference)
import functools

import jax
import jax.numpy as jnp
from jax import lax
from jax.experimental import pallas as pl
from jax.experimental.pallas import tpu as pltpu
from jax.experimental.pallas import tpu_sc as plsc

N_NODES = 10000
N_EDGES = 320000
D = 128

NC = 2
NS = 16
NW = NC * NS
CHUNK = 80
NR = 3
NI = 4
NCHUNK = 125
STEADY_END = 2 + 12 * ((NCHUNK - 4) // 12)
ZB = 128
ACC_ROWS = N_NODES
ZROWS_A = 624
ZROWS_B = ACC_ROWS - 15 * ZROWS_A


def _sc_segsum(x, src_flat, dst_flat, zblk):
    mesh = plsc.VectorSubcoreMesh(core_axis_name="c", subcore_axis_name="s")

    @functools.partial(
        pl.kernel,
        out_type=jax.ShapeDtypeStruct((NC, ACC_ROWS, D), jnp.float32),
        mesh=mesh,
        scratch_types=[
            pltpu.VMEM_SHARED((ACC_ROWS, D), jnp.float32),
            pltpu.VMEM((NI, CHUNK), jnp.int32),
            pltpu.VMEM((NI, CHUNK), jnp.int32),
            pltpu.VMEM((NR, CHUNK, D), jnp.float32),
            pltpu.SemaphoreType.DMA((NI,)),
            pltpu.SemaphoreType.DMA((NR,)),
            pltpu.SemaphoreType.DMA((NR,)),
            pltpu.SemaphoreType.DMA,
        ],
    )
    def k(x_hbm, src_hbm, dst_hbm, z_hbm, out_hbm,
          acc_sh, sidx, didx, rows_v, isem, gsem, ssem, zsem):
        cid = lax.axis_index("c")
        sid = lax.axis_index("s")
        wid = cid * NS + sid
        base = wid * (NCHUNK * CHUNK)

        def i_start(j, s):
            off = base + j * CHUNK
            pltpu.async_copy(src_hbm.at[pl.ds(off, CHUNK)], sidx.at[s],
                             isem.at[s])
            pltpu.async_copy(dst_hbm.at[pl.ds(off, CHUNK)], didx.at[s],
                             isem.at[s])

        def i_wait(j, s):
            off = base + j * CHUNK
            pltpu.make_async_copy(src_hbm.at[pl.ds(off, CHUNK)], sidx.at[s],
                                  isem.at[s]).wait()
            pltpu.make_async_copy(dst_hbm.at[pl.ds(off, CHUNK)], didx.at[s],
                                  isem.at[s]).wait()

        def g_start(s, b):
            pltpu.async_copy(x_hbm.at[sidx.at[s]], rows_v.at[b], gsem.at[b])

        def g_wait(s, b):
            pltpu.make_async_copy(
                x_hbm.at[sidx.at[s]], rows_v.at[b], gsem.at[b]).wait()

        def s_start(s, b):
            pltpu.async_copy(rows_v.at[b], acc_sh.at[didx.at[s]],
                             ssem.at[b], add=True)

        def s_wait(s, b):
            pltpu.make_async_copy(rows_v.at[b], acc_sh.at[didx.at[s]],
                                  ssem.at[b]).wait()

        i_start(0, 0)
        i_start(1, 1)

        @pl.when(sid < 15)
        def _():
            for r in range(4):
                pltpu.async_copy(z_hbm,
                                 acc_sh.at[pl.ds(sid * ZROWS_A + r * ZB,
                                                 ZB)], zsem)
            pltpu.async_copy(z_hbm.at[pl.ds(0, ZROWS_A - 4 * ZB)],
                             acc_sh.at[pl.ds(sid * ZROWS_A + 4 * ZB,
                                             ZROWS_A - 4 * ZB)], zsem)

        @pl.when(sid == 15)
        def _():
            for r in range(5):
                pltpu.async_copy(z_hbm,
                                 acc_sh.at[pl.ds(15 * ZROWS_A + r * ZB,
                                                 ZB)], zsem)
        for r in range(4):
            pltpu.make_async_copy(z_hbm,
                                  acc_sh.at[pl.ds(0, ZB)], zsem).wait()

        @pl.when(sid < 15)
        def _():
            pltpu.make_async_copy(
                z_hbm.at[pl.ds(0, ZROWS_A - 4 * ZB)],
                acc_sh.at[pl.ds(0, ZROWS_A - 4 * ZB)], zsem).wait()

        @pl.when(sid == 15)
        def _():
            pltpu.make_async_copy(z_hbm,
                                  acc_sh.at[pl.ds(0, ZB)], zsem).wait()

        plsc.subcore_barrier()

        def step(j, jm, full):
            if full:
                s_wait((jm - 2) % NI, (jm - 2) % NR)
            i_start(j + 2, (jm + 2) % NI)
            i_wait(j + 1, (jm + 1) % NI)
            g_start((jm + 1) % NI, (jm + 1) % NR)
            g_wait(jm % NI, jm % NR)
            s_start(jm % NI, jm % NR)

        i_wait(0, 0)
        g_start(0, 0)
        step(0, 0, full=False)
        step(1, 1, full=False)

        @pl.loop(2, STEADY_END, step=12)
        def _(j0):
            for d in range(12):
                step(j0 + d, 2 + d, full=True)

        for j in range(STEADY_END, NCHUNK):
            s_wait((j - 2) % NI, (j - 2) % NR)
            if j + 2 <= NCHUNK - 1:
                i_start(j + 2, (j + 2) % NI)
            if j + 1 <= NCHUNK - 1:
                i_wait(j + 1, (j + 1) % NI)
                g_start((j + 1) % NI, (j + 1) % NR)
            g_wait(j % NI, j % NR)
            s_start(j % NI, j % NR)
        s_wait((NCHUNK - 2) % NI, (NCHUNK - 2) % NR)
        s_wait((NCHUNK - 1) % NI, (NCHUNK - 1) % NR)

        plsc.subcore_barrier()

        @pl.when(sid < 15)
        def _():
            pltpu.sync_copy(acc_sh.at[pl.ds(sid * ZROWS_A, ZROWS_A)],
                            out_hbm.at[cid, pl.ds(sid * ZROWS_A, ZROWS_A)])

        @pl.when(sid == 15)
        def _():
            pltpu.sync_copy(acc_sh.at[pl.ds(15 * ZROWS_A, ZROWS_B)],
                            out_hbm.at[cid, pl.ds(15 * ZROWS_A, ZROWS_B)])

    return k(x, src_flat, dst_flat, zblk)


_BR = 2000


def _tc_stage1(p, W1):
    def body(p_ref, w_ref, o_ref):
        s = p_ref[0] + p_ref[1]
        o_ref[...] = jnp.maximum(
            jnp.dot(s, w_ref[...], preferred_element_type=jnp.float32), 0.0)

    return pl.pallas_call(
        body,
        grid=(N_NODES // _BR,),
        in_specs=[
            pl.BlockSpec((NC, _BR, D), lambda i: (0, i, 0)),
            pl.BlockSpec((D, D), lambda i: (0, 0)),
        ],
        out_specs=pl.BlockSpec((_BR, D), lambda i: (i, 0)),
        out_shape=jax.ShapeDtypeStruct((N_NODES, D), jnp.float32),
    )(p, W1)


def _tc_stage2(h1, p2, W2, M1, b1r, M2, b2r):
    def body(h1_ref, p_ref, w2_ref, m1_ref, b1_ref, m2_ref, b2_ref, o_ref):
        agg2 = p_ref[0] + p_ref[1]
        h2 = jnp.maximum(
            jnp.dot(agg2, w2_ref[...], preferred_element_type=jnp.float32), 0.0)
        cat = jnp.concatenate([h1_ref[...], h2], axis=-1)
        z = jnp.dot(cat, m1_ref[...], preferred_element_type=jnp.float32)
        z = jnp.maximum(z + b1_ref[...], 0.0)
        o_ref[...] = (jnp.dot(z, m2_ref[...],
                              preferred_element_type=jnp.float32)
                      + b2_ref[...])

    return pl.pallas_call(
        body,
        grid=(N_NODES // _BR,),
        in_specs=[
            pl.BlockSpec((_BR, D), lambda i: (i, 0)),
            pl.BlockSpec((NC, _BR, D), lambda i: (0, i, 0)),
            pl.BlockSpec((D, D), lambda i: (0, 0)),
            pl.BlockSpec((2 * D, 64), lambda i: (0, 0)),
            pl.BlockSpec((1, 64), lambda i: (0, 0)),
            pl.BlockSpec((64, 1), lambda i: (0, 0)),
            pl.BlockSpec((1, 1), lambda i: (0, 0)),
        ],
        out_specs=pl.BlockSpec((_BR, 1), lambda i: (i, 0)),
        out_shape=jax.ShapeDtypeStruct((N_NODES, 1), jnp.float32),
    )(h1, p2, W2, M1, b1r, M2, b2r)


def kernel(x, edge_index, W1, W2, M1, b1, M2, b2):
    src = edge_index[0].astype(jnp.int32)
    dst = edge_index[1].astype(jnp.int32)
    zblk = jnp.zeros((ZB, D), jnp.float32)

    p1 = _sc_segsum(x, src, dst, zblk)
    h1 = _tc_stage1(p1, W1)
    p2 = _sc_segsum(h1, src, dst, zblk)
    return _tc_stage2(h1, p2, W2, M1, b1.reshape(1, 64),
                      M2, b2.reshape(1, 1))


# speedup vs baseline: 1.0387x; 1.0387x over previous
import functools

import jax
import jax.numpy as jnp
from jax import lax
from jax.experimental import pallas as pl
from jax.experimental.pallas import tpu as pltpu
from jax.experimental.pallas import tpu_sc as plsc

N_NODES = 10000
N_EDGES = 320000
D = 128

NC = 2
NS = 16
NW = NC * NS
CHUNK = 128
NR = 3
NI = 4
NCHUNK = 78
ACC_ROWS = N_NODES
ZROWS_A = 624
ZROWS_B = ACC_ROWS - 15 * ZROWS_A


def _sc_segsum(x, src_flat, dst_flat, zblk):
    mesh = plsc.VectorSubcoreMesh(core_axis_name="c", subcore_axis_name="s")

    @functools.partial(
        pl.kernel,
        out_type=jax.ShapeDtypeStruct((NC, ACC_ROWS, D), jnp.float32),
        mesh=mesh,
        scratch_types=[
            pltpu.VMEM_SHARED((ACC_ROWS, D), jnp.float32),
            pltpu.VMEM((NI, CHUNK), jnp.int32),
            pltpu.VMEM((NI, CHUNK), jnp.int32),
            pltpu.VMEM((NR, CHUNK, D), jnp.float32),
            pltpu.SemaphoreType.DMA((NI,)),
            pltpu.SemaphoreType.DMA((NR,)),
            pltpu.SemaphoreType.DMA((NR,)),
            pltpu.SemaphoreType.DMA,
        ],
    )
    def k(x_hbm, src_hbm, dst_hbm, z_hbm, out_hbm,
          acc_sh, sidx, didx, rows_v, isem, gsem, ssem, zsem):
        cid = lax.axis_index("c")
        sid = lax.axis_index("s")
        wid = cid * NS + sid
        base = (wid * (NCHUNK * CHUNK)
                + jnp.minimum(wid, 4) * CHUNK)

        def i_start(j, s):
            off = base + j * CHUNK
            pltpu.async_copy(src_hbm.at[pl.ds(off, CHUNK)], sidx.at[s],
                             isem.at[s])
            pltpu.async_copy(dst_hbm.at[pl.ds(off, CHUNK)], didx.at[s],
                             isem.at[s])

        def i_wait(j, s):
            off = base + j * CHUNK
            pltpu.make_async_copy(src_hbm.at[pl.ds(off, CHUNK)], sidx.at[s],
                                  isem.at[s]).wait()
            pltpu.make_async_copy(dst_hbm.at[pl.ds(off, CHUNK)], didx.at[s],
                                  isem.at[s]).wait()

        def g_start(s, b):
            pltpu.async_copy(x_hbm.at[sidx.at[s]], rows_v.at[b], gsem.at[b])

        def g_wait(s, b):
            pltpu.make_async_copy(
                x_hbm.at[sidx.at[s]], rows_v.at[b], gsem.at[b]).wait()

        def s_start(s, b):
            pltpu.async_copy(rows_v.at[b], acc_sh.at[didx.at[s]],
                             ssem.at[b], add=True)

        def s_wait(s, b):
            pltpu.make_async_copy(rows_v.at[b], acc_sh.at[didx.at[s]],
                                  ssem.at[b]).wait()

        i_start(0, 0)
        i_start(1, 1)

        pltpu.sync_copy(z_hbm, rows_v.at[0])

        @pl.when(sid < 15)
        def _():
            for r in range(4):
                pltpu.async_copy(rows_v.at[0],
                                 acc_sh.at[pl.ds(sid * ZROWS_A + r * CHUNK,
                                                 CHUNK)], zsem)
            pltpu.async_copy(rows_v.at[0].at[pl.ds(0, ZROWS_A - 4 * CHUNK)],
                             acc_sh.at[pl.ds(sid * ZROWS_A + 4 * CHUNK,
                                             ZROWS_A - 4 * CHUNK)], zsem)

        @pl.when(sid == 15)
        def _():
            for r in range(5):
                pltpu.async_copy(rows_v.at[0],
                                 acc_sh.at[pl.ds(15 * ZROWS_A + r * CHUNK,
                                                 CHUNK)], zsem)
        for r in range(4):
            pltpu.make_async_copy(rows_v.at[0],
                                  acc_sh.at[pl.ds(0, CHUNK)], zsem).wait()

        @pl.when(sid < 15)
        def _():
            pltpu.make_async_copy(
                rows_v.at[0].at[pl.ds(0, ZROWS_A - 4 * CHUNK)],
                acc_sh.at[pl.ds(0, ZROWS_A - 4 * CHUNK)], zsem).wait()

        @pl.when(sid == 15)
        def _():
            pltpu.make_async_copy(rows_v.at[0],
                                  acc_sh.at[pl.ds(0, CHUNK)], zsem).wait()

        plsc.subcore_barrier()

        def step(j, jm, full):
            if full:
                s_wait((jm - 2) % NI, (jm - 2) % NR)
            i_start(j + 2, (jm + 2) % NI)
            i_wait(j + 1, (jm + 1) % NI)
            g_start((jm + 1) % NI, (jm + 1) % NR)
            g_wait(jm % NI, jm % NR)
            s_start(jm % NI, jm % NR)

        i_wait(0, 0)
        g_start(0, 0)
        step(0, 0, full=False)
        step(1, 1, full=False)

        @pl.loop(2, 74, step=12)
        def _(j0):
            for d in range(12):
                step(j0 + d, 2 + d, full=True)

        for j in range(74, NCHUNK):
            s_wait((j - 2) % NI, (j - 2) % NR)
            if j + 2 <= NCHUNK - 1:
                i_start(j + 2, (j + 2) % NI)
            if j + 1 <= NCHUNK - 1:
                i_wait(j + 1, (j + 1) % NI)
                g_start((j + 1) % NI, (j + 1) % NR)
            g_wait(j % NI, j % NR)
            s_start(j % NI, j % NR)
        s_wait((NCHUNK - 2) % NI, (NCHUNK - 2) % NR)
        s_wait((NCHUNK - 1) % NI, (NCHUNK - 1) % NR)

        @pl.when(wid < 4)
        def _():
            off = base + NCHUNK * CHUNK
            pltpu.sync_copy(src_hbm.at[pl.ds(off, CHUNK)], sidx.at[0])
            pltpu.sync_copy(dst_hbm.at[pl.ds(off, CHUNK)], didx.at[0])
            pltpu.sync_copy(x_hbm.at[sidx.at[0]], rows_v.at[0])
            pltpu.sync_copy(rows_v.at[0], acc_sh.at[didx.at[0]], add=True)

        plsc.subcore_barrier()

        @pl.when(sid < 15)
        def _():
            pltpu.sync_copy(acc_sh.at[pl.ds(sid * ZROWS_A, ZROWS_A)],
                            out_hbm.at[cid, pl.ds(sid * ZROWS_A, ZROWS_A)])

        @pl.when(sid == 15)
        def _():
            pltpu.sync_copy(acc_sh.at[pl.ds(15 * ZROWS_A, ZROWS_B)],
                            out_hbm.at[cid, pl.ds(15 * ZROWS_A, ZROWS_B)])

    return k(x, src_flat, dst_flat, zblk)


_BR = 2000


def _tc_stage1(p, W1):
    def body(p_ref, w_ref, o_ref):
        s = p_ref[0] + p_ref[1]
        o_ref[...] = jnp.maximum(
            jnp.dot(s, w_ref[...], preferred_element_type=jnp.float32), 0.0)

    return pl.pallas_call(
        body,
        grid=(N_NODES // _BR,),
        in_specs=[
            pl.BlockSpec((NC, _BR, D), lambda i: (0, i, 0)),
            pl.BlockSpec((D, D), lambda i: (0, 0)),
        ],
        out_specs=pl.BlockSpec((_BR, D), lambda i: (i, 0)),
        out_shape=jax.ShapeDtypeStruct((N_NODES, D), jnp.float32),
    )(p, W1)


def _tc_stage2(h1, p2, W2, M1, b1r, M2, b2r):
    def body(h1_ref, p_ref, w2_ref, m1_ref, b1_ref, m2_ref, b2_ref, o_ref):
        agg2 = p_ref[0] + p_ref[1]
        h2 = jnp.maximum(
            jnp.dot(agg2, w2_ref[...], preferred_element_type=jnp.float32), 0.0)
        cat = jnp.concatenate([h1_ref[...], h2], axis=-1)
        z = jnp.dot(cat, m1_ref[...], preferred_element_type=jnp.float32)
        z = jnp.maximum(z + b1_ref[...], 0.0)
        o_ref[...] = (jnp.dot(z, m2_ref[...],
                              preferred_element_type=jnp.float32)
                      + b2_ref[...])

    return pl.pallas_call(
        body,
        grid=(N_NODES // _BR,),
        in_specs=[
            pl.BlockSpec((_BR, D), lambda i: (i, 0)),
            pl.BlockSpec((NC, _BR, D), lambda i: (0, i, 0)),
            pl.BlockSpec((D, D), lambda i: (0, 0)),
            pl.BlockSpec((2 * D, 64), lambda i: (0, 0)),
            pl.BlockSpec((1, 64), lambda i: (0, 0)),
            pl.BlockSpec((64, 1), lambda i: (0, 0)),
            pl.BlockSpec((1, 1), lambda i: (0, 0)),
        ],
        out_specs=pl.BlockSpec((_BR, 1), lambda i: (i, 0)),
        out_shape=jax.ShapeDtypeStruct((N_NODES, 1), jnp.float32),
    )(h1, p2, W2, M1, b1r, M2, b2r)


def kernel(x, edge_index, W1, W2, M1, b1, M2, b2):
    src = edge_index[0].astype(jnp.int32)
    dst = edge_index[1].astype(jnp.int32)
    zblk = jnp.zeros((CHUNK, D), jnp.float32)

    p1 = _sc_segsum(x, src, dst, zblk)
    h1 = _tc_stage1(p1, W1)
    p2 = _sc_segsum(h1, src, dst, zblk)
    return _tc_stage2(h1, p2, W2, M1, b1.reshape(1, 64),
                      M2, b2.reshape(1, 1))


# speedup vs baseline: 1.0804x; 1.0401x over previous
import functools

import jax
import jax.numpy as jnp
from jax import lax
from jax.experimental import pallas as pl
from jax.experimental.pallas import tpu as pltpu
from jax.experimental.pallas import tpu_sc as plsc

N_NODES = 10000
N_EDGES = 320000
D = 128

NC = 2
NS = 16
NW = NC * NS
CHUNK = 128
NR = 3
NI = 4
NCHUNK = 78
ACC_ROWS = N_NODES
ZROWS_A = 624
ZROWS_B = ACC_ROWS - 15 * ZROWS_A


def _sc_segsum(x, ei_flat, zblk):
    mesh = plsc.VectorSubcoreMesh(core_axis_name="c", subcore_axis_name="s")

    @functools.partial(
        pl.kernel,
        out_type=jax.ShapeDtypeStruct((NC, ACC_ROWS, D), jnp.float32),
        mesh=mesh,
        scratch_types=[
            pltpu.VMEM_SHARED((ACC_ROWS, D), jnp.float32),
            pltpu.VMEM((NI, CHUNK), jnp.int32),
            pltpu.VMEM((NI, CHUNK), jnp.int32),
            pltpu.VMEM((NR, CHUNK, D), jnp.float32),
            pltpu.SemaphoreType.DMA((NI,)),
            pltpu.SemaphoreType.DMA((NR,)),
            pltpu.SemaphoreType.DMA((NR,)),
            pltpu.SemaphoreType.DMA,
        ],
    )
    def k(x_hbm, e_hbm, z_hbm, out_hbm,
          acc_sh, sidx, didx, rows_v, isem, gsem, ssem, zsem):
        cid = lax.axis_index("c")
        sid = lax.axis_index("s")
        wid = cid * NS + sid
        base = (wid * (NCHUNK * CHUNK)
                + jnp.minimum(wid, 4) * CHUNK)

        def i_start(j, s):
            off = base + j * CHUNK
            pltpu.async_copy(e_hbm.at[pl.ds(off, CHUNK)], sidx.at[s],
                             isem.at[s])
            pltpu.async_copy(e_hbm.at[pl.ds(N_EDGES + off, CHUNK)],
                             didx.at[s], isem.at[s])

        def i_wait(j, s):
            off = base + j * CHUNK
            pltpu.make_async_copy(e_hbm.at[pl.ds(off, CHUNK)], sidx.at[s],
                                  isem.at[s]).wait()
            pltpu.make_async_copy(e_hbm.at[pl.ds(N_EDGES + off, CHUNK)],
                                  didx.at[s], isem.at[s]).wait()

        def g_start(s, b):
            pltpu.async_copy(x_hbm.at[sidx.at[s]], rows_v.at[b], gsem.at[b])

        def g_wait(s, b):
            pltpu.make_async_copy(
                x_hbm.at[sidx.at[s]], rows_v.at[b], gsem.at[b]).wait()

        def s_start(s, b):
            pltpu.async_copy(rows_v.at[b], acc_sh.at[didx.at[s]],
                             ssem.at[b], add=True)

        def s_wait(s, b):
            pltpu.make_async_copy(rows_v.at[b], acc_sh.at[didx.at[s]],
                                  ssem.at[b]).wait()

        i_start(0, 0)
        i_start(1, 1)

        pltpu.sync_copy(z_hbm, rows_v.at[0])

        @pl.when(sid < 15)
        def _():
            for r in range(4):
                pltpu.async_copy(rows_v.at[0],
                                 acc_sh.at[pl.ds(sid * ZROWS_A + r * CHUNK,
                                                 CHUNK)], zsem)
            pltpu.async_copy(rows_v.at[0].at[pl.ds(0, ZROWS_A - 4 * CHUNK)],
                             acc_sh.at[pl.ds(sid * ZROWS_A + 4 * CHUNK,
                                             ZROWS_A - 4 * CHUNK)], zsem)

        @pl.when(sid == 15)
        def _():
            for r in range(5):
                pltpu.async_copy(rows_v.at[0],
                                 acc_sh.at[pl.ds(15 * ZROWS_A + r * CHUNK,
                                                 CHUNK)], zsem)
        for r in range(4):
            pltpu.make_async_copy(rows_v.at[0],
                                  acc_sh.at[pl.ds(0, CHUNK)], zsem).wait()

        @pl.when(sid < 15)
        def _():
            pltpu.make_async_copy(
                rows_v.at[0].at[pl.ds(0, ZROWS_A - 4 * CHUNK)],
                acc_sh.at[pl.ds(0, ZROWS_A - 4 * CHUNK)], zsem).wait()

        @pl.when(sid == 15)
        def _():
            pltpu.make_async_copy(rows_v.at[0],
                                  acc_sh.at[pl.ds(0, CHUNK)], zsem).wait()

        plsc.subcore_barrier()

        def step(j, jm, full):
            if full:
                s_wait((jm - 2) % NI, (jm - 2) % NR)
            i_start(j + 2, (jm + 2) % NI)
            i_wait(j + 1, (jm + 1) % NI)
            g_start((jm + 1) % NI, (jm + 1) % NR)
            g_wait(jm % NI, jm % NR)
            s_start(jm % NI, jm % NR)

        i_wait(0, 0)
        g_start(0, 0)
        step(0, 0, full=False)
        step(1, 1, full=False)

        @pl.loop(2, 74, step=12)
        def _(j0):
            for d in range(12):
                step(j0 + d, 2 + d, full=True)

        for j in range(74, NCHUNK):
            s_wait((j - 2) % NI, (j - 2) % NR)
            if j + 2 <= NCHUNK - 1:
                i_start(j + 2, (j + 2) % NI)
            if j + 1 <= NCHUNK - 1:
                i_wait(j + 1, (j + 1) % NI)
                g_start((j + 1) % NI, (j + 1) % NR)
            g_wait(j % NI, j % NR)
            s_start(j % NI, j % NR)
        s_wait((NCHUNK - 2) % NI, (NCHUNK - 2) % NR)
        s_wait((NCHUNK - 1) % NI, (NCHUNK - 1) % NR)

        @pl.when(wid < 4)
        def _():
            off = base + NCHUNK * CHUNK
            pltpu.sync_copy(e_hbm.at[pl.ds(off, CHUNK)], sidx.at[0])
            pltpu.sync_copy(e_hbm.at[pl.ds(N_EDGES + off, CHUNK)], didx.at[0])
            pltpu.sync_copy(x_hbm.at[sidx.at[0]], rows_v.at[0])
            pltpu.sync_copy(rows_v.at[0], acc_sh.at[didx.at[0]], add=True)

        plsc.subcore_barrier()

        @pl.when(sid < 15)
        def _():
            pltpu.sync_copy(acc_sh.at[pl.ds(sid * ZROWS_A, ZROWS_A)],
                            out_hbm.at[cid, pl.ds(sid * ZROWS_A, ZROWS_A)])

        @pl.when(sid == 15)
        def _():
            pltpu.sync_copy(acc_sh.at[pl.ds(15 * ZROWS_A, ZROWS_B)],
                            out_hbm.at[cid, pl.ds(15 * ZROWS_A, ZROWS_B)])

    return k(x, ei_flat, zblk)


_BR = 2000


def _tc_stage1(p, W1):
    def body(p_ref, w_ref, o_ref):
        s = p_ref[0] + p_ref[1]
        o_ref[...] = jnp.maximum(
            jnp.dot(s, w_ref[...], preferred_element_type=jnp.float32), 0.0)

    return pl.pallas_call(
        body,
        grid=(N_NODES // _BR,),
        in_specs=[
            pl.BlockSpec((NC, _BR, D), lambda i: (0, i, 0)),
            pl.BlockSpec((D, D), lambda i: (0, 0)),
        ],
        out_specs=pl.BlockSpec((_BR, D), lambda i: (i, 0)),
        out_shape=jax.ShapeDtypeStruct((N_NODES, D), jnp.float32),
    )(p, W1)


def _tc_stage2(h1, p2, W2, M1, b1r, M2, b2r):
    def body(h1_ref, p_ref, w2_ref, m1_ref, b1_ref, m2_ref, b2_ref, o_ref):
        agg2 = p_ref[0] + p_ref[1]
        h2 = jnp.maximum(
            jnp.dot(agg2, w2_ref[...], preferred_element_type=jnp.float32), 0.0)
        cat = jnp.concatenate([h1_ref[...], h2], axis=-1)
        z = jnp.dot(cat, m1_ref[...], preferred_element_type=jnp.float32)
        z = jnp.maximum(z + b1_ref[...], 0.0)
        o_ref[...] = (jnp.dot(z, m2_ref[...],
                              preferred_element_type=jnp.float32)
                      + b2_ref[...])

    return pl.pallas_call(
        body,
        grid=(N_NODES // _BR,),
        in_specs=[
            pl.BlockSpec((_BR, D), lambda i: (i, 0)),
            pl.BlockSpec((NC, _BR, D), lambda i: (0, i, 0)),
            pl.BlockSpec((D, D), lambda i: (0, 0)),
            pl.BlockSpec((2 * D, 64), lambda i: (0, 0)),
            pl.BlockSpec((1, 64), lambda i: (0, 0)),
            pl.BlockSpec((64, 1), lambda i: (0, 0)),
            pl.BlockSpec((1, 1), lambda i: (0, 0)),
        ],
        out_specs=pl.BlockSpec((_BR, 1), lambda i: (i, 0)),
        out_shape=jax.ShapeDtypeStruct((N_NODES, 1), jnp.float32),
    )(h1, p2, W2, M1, b1r, M2, b2r)


def kernel(x, edge_index, W1, W2, M1, b1, M2, b2):
    ei_flat = edge_index.astype(jnp.int32).reshape(2 * N_EDGES)
    zblk = jnp.zeros((CHUNK, D), jnp.float32)

    p1 = _sc_segsum(x, ei_flat, zblk)
    h1 = _tc_stage1(p1, W1)
    p2 = _sc_segsum(h1, ei_flat, zblk)
    return _tc_stage2(h1, p2, W2, M1, b1.reshape(1, 64),
                      M2, b2.reshape(1, 1))
